# baseline (speedup 1.0000x reference)
import functools

import jax
import jax.numpy as jnp
from jax import lax
from jax.experimental import pallas as pl
from jax.experimental.pallas import tpu as pltpu
from jax.experimental.pallas import tpu_sc as plsc

_NUM_SAMPLES = 16
_SAMPLE_RATE = 4


def _sample_indices(t):
    sample_range = _NUM_SAMPLES * _SAMPLE_RATE
    sample_pos = max(1, 1 + t - sample_range)
    start_idx = 0 if sample_pos == 1 else sample_pos // 2
    idx = [((i * _SAMPLE_RATE + start_idx) % t) + 1 for i in range(_NUM_SAMPLES)]
    return [min(max(v, 0), t - 1) for v in idx]


def kernel(x):
    c, t, h, w = x.shape
    idxs = _sample_indices(t)
    a0 = idxs[0]
    step = idxs[1] - idxs[0]
    assert all(idxs[j] == a0 + step * j for j in range(_NUM_SAMPLES))
    assert _NUM_SAMPLES == 16

    frame = h * w
    n_frames = c * _NUM_SAMPLES
    half = frame // 2
    total_chunks = n_frames * 2

    info = plsc.get_sparse_core_info()
    nc, ns = info.num_cores, info.num_subcores
    nw = nc * ns
    per_w = total_chunks // nw
    assert total_chunks % nw == 0 and frame % 2 == 0 and half % 8 == 0

    mesh = plsc.VectorSubcoreMesh(core_axis_name="c", subcore_axis_name="s")

    @functools.partial(
        pl.kernel,
        out_type=jax.ShapeDtypeStruct((n_frames * frame,), x.dtype),
        mesh=mesh,
        scratch_types=[pltpu.SemaphoreType.DMA] * per_w,
    )
    def temporal_gather(x_hbm, out_hbm, *sems):
        wid = lax.axis_index("s") * nc + lax.axis_index("c")
        k0 = wid * per_w
        copies = []
        for u in range(per_w):
            k = k0 + u
            f = k >> 1
            hh = k & 1
            ci = f >> 4
            j = f & (_NUM_SAMPLES - 1)
            src = (ci * t + a0 + step * j) * frame + hh * half
            dst = k * half
            copies.append(
                pltpu.async_copy(
                    x_hbm.at[pl.ds(src, half)],
                    out_hbm.at[pl.ds(dst, half)],
                    sems[u],
                )
            )
        for cp in copies:
            cp.wait()

    out = temporal_gather(x.reshape(-1))
    return out.reshape(c, _NUM_SAMPLES, h, w)


# speedup vs baseline: 1.1894x; 1.1894x over previous
import functools

import jax
import jax.numpy as jnp
from jax import lax
from jax.experimental import pallas as pl
from jax.experimental.pallas import tpu as pltpu
from jax.experimental.pallas import tpu_sc as plsc

_NUM_SAMPLES = 16
_SAMPLE_RATE = 4


def _sample_indices(t):
    sample_range = _NUM_SAMPLES * _SAMPLE_RATE
    sample_pos = max(1, 1 + t - sample_range)
    start_idx = 0 if sample_pos == 1 else sample_pos // 2
    idx = [((i * _SAMPLE_RATE + start_idx) % t) + 1 for i in range(_NUM_SAMPLES)]
    return [min(max(v, 0), t - 1) for v in idx]


def kernel(x):
    c, t, h, w = x.shape
    idxs = _sample_indices(t)
    a0 = idxs[0]
    step = idxs[1] - idxs[0]
    assert all(idxs[j] == a0 + step * j for j in range(_NUM_SAMPLES))
    assert _NUM_SAMPLES == 16

    n_frames = c * _NUM_SAMPLES
    half = h // 2
    total_chunks = n_frames * 2

    info = plsc.get_sparse_core_info()
    nc, ns = info.num_cores, info.num_subcores
    nw = nc * ns
    per_w = total_chunks // nw
    assert total_chunks % nw == 0 and h % 2 == 0 and half % 8 == 0

    mesh = plsc.VectorSubcoreMesh(core_axis_name="c", subcore_axis_name="s")

    @functools.partial(
        pl.kernel,
        out_type=jax.ShapeDtypeStruct((total_chunks, half, w), x.dtype),
        mesh=mesh,
        scratch_types=[pltpu.SemaphoreType.DMA] * per_w,
        compiler_params=pltpu.CompilerParams(use_tc_tiling_on_sc=True),
    )
    def temporal_gather(x_hbm, out_hbm, *sems):
        wid = lax.axis_index("s") * nc + lax.axis_index("c")
        k0 = wid * per_w
        copies = []
        for u in range(per_w):
            k = k0 + u
            f = k >> 1
            hh = k & 1
            ci = f >> 4
            j = f & (_NUM_SAMPLES - 1)
            src = ((ci * t + a0 + step * j) << 1) | hh
            copies.append(
                pltpu.async_copy(x_hbm.at[src], out_hbm.at[k], sems[u])
            )
        for cp in copies:
            cp.wait()

    out = temporal_gather(x.reshape(c * t * 2, half, w))
    return out.reshape(c, _NUM_SAMPLES, h, w)


# speedup vs baseline: 5.5030x; 4.6266x over previous
import functools

import jax
import jax.numpy as jnp
from jax import lax
from jax.experimental import pallas as pl
from jax.experimental.pallas import tpu as pltpu
from jax.experimental.pallas import tpu_sc as plsc

_NUM_SAMPLES = 16
_SAMPLE_RATE = 4


def _sample_indices(t):
    sample_range = _NUM_SAMPLES * _SAMPLE_RATE
    sample_pos = max(1, 1 + t - sample_range)
    start_idx = 0 if sample_pos == 1 else sample_pos // 2
    idx = [((i * _SAMPLE_RATE + start_idx) % t) + 1 for i in range(_NUM_SAMPLES)]
    return [min(max(v, 0), t - 1) for v in idx]


def kernel(x):
    c, t, h, w = x.shape
    idxs = _sample_indices(t)
    a0 = idxs[0]
    step = idxs[1] - idxs[0]
    assert all(idxs[j] == a0 + step * j for j in range(_NUM_SAMPLES))
    assert _NUM_SAMPLES == 16

    n_frames = c * _NUM_SAMPLES
    half = h // 2
    total_chunks = n_frames * 2

    info = plsc.get_sparse_core_info()
    nc, ns = info.num_cores, info.num_subcores
    nw = nc * ns
    per_w = total_chunks // nw
    assert total_chunks % nw == 0 and h % 2 == 0 and half % 8 == 0

    mesh = plsc.VectorSubcoreMesh(core_axis_name="c", subcore_axis_name="s")

    @functools.partial(
        pl.kernel,
        out_type=jax.ShapeDtypeStruct((total_chunks, half, w), x.dtype),
        mesh=mesh,
        scratch_types=(
            [pltpu.VMEM((half, w), jnp.float32)] * per_w
            + [pltpu.SemaphoreType.DMA] * (2 * per_w)
        ),
        compiler_params=pltpu.CompilerParams(use_tc_tiling_on_sc=True),
    )
    def temporal_gather(x_hbm, out_hbm, *args):
        bufs = args[:per_w]
        gsems = args[per_w : 2 * per_w]
        ssems = args[2 * per_w :]
        wid = lax.axis_index("s") * nc + lax.axis_index("c")
        k0 = wid * per_w
        gathers = []
        for u in range(per_w):
            k = k0 + u
            f = k >> 1
            hh = k & 1
            ci = f >> 4
            j = f & (_NUM_SAMPLES - 1)
            src = ((ci * t + a0 + step * j) << 1) | hh
            gathers.append(pltpu.async_copy(x_hbm.at[src], bufs[u], gsems[u]))
        scatters = []
        for u in range(per_w):
            k = k0 + u
            gathers[u].wait()
            scatters.append(pltpu.async_copy(bufs[u], out_hbm.at[k], ssems[u]))
        for cp in scatters:
            cp.wait()

    out = temporal_gather(x.reshape(c * t * 2, half, w))
    return out.reshape(c, _NUM_SAMPLES, h, w)
